# baseline (speedup 1.0000x reference)
import functools

import jax
import jax.numpy as jnp
from jax import lax
from jax.experimental import pallas as pl
from jax.experimental.pallas import tpu as pltpu
from jax.experimental.pallas import tpu_sc as plsc

_B, _C, _W, _H = 32, 19, 128, 128
_NC, _NS = 2, 16
_JB = 16
_NCH = _W // _JB
_OPI = _JB * _C
_OST = _OPI + 1


def _sc_body(x_hbm, out_hbm, in_v, out_v):
    b = lax.axis_index("s") * _NC + lax.axis_index("c")
    iota = lax.iota(jnp.int32, 16)
    for ch in range(_NCH):
        j0 = ch * _JB
        pltpu.sync_copy(x_hbm.at[b, :, pl.ds(j0, _JB), :], in_v)

        def body(t, carry):
            j = t >> 3
            i0 = (t & 7) * 16
            row = iota + i0
            vecs = [in_v[c, j, pl.ds(i0, 16)] for c in range(_C)]
            s = vecs[0]
            for c in range(1, _C):
                s = s + vecs[c]
            r = 1.0 / s
            colbase = j * _C
            for c in range(_C):
                col = jnp.full((16,), colbase + c, jnp.int32)
                plsc.store_scatter(out_v, [row, col], vecs[c] * r)
            return carry

        lax.fori_loop(0, _JB * (_H // 16), body, 0)
        pltpu.sync_copy(out_v.at[:, pl.ds(0, _OPI)],
                        out_hbm.at[b, :, pl.ds(j0 * _C, _OPI)])


def kernel(x):
    k = pl.kernel(
        _sc_body,
        out_type=jax.ShapeDtypeStruct((_B, _H, _W * _C), jnp.float32),
        mesh=plsc.VectorSubcoreMesh(core_axis_name="c", subcore_axis_name="s"),
        compiler_params=pltpu.CompilerParams(
            use_tc_tiling_on_sc=False, needs_layout_passes=False),
        scratch_types=[
            pltpu.VMEM((_C, _JB, _H), jnp.float32),
            pltpu.VMEM((_H, _OST), jnp.float32),
        ],
    )
    out3 = k(x)
    return out3.reshape(_B * _W * _H, _C)


# speedup vs baseline: 14.0263x; 14.0263x over previous
import jax
import jax.numpy as jnp
from jax.experimental import pallas as pl
from jax.experimental.pallas import tpu as pltpu

_B, _C, _W, _H = 32, 19, 128, 128
_PIX = _W * _H


def _body(x_ref, o_ref):
    data = x_ref[0]
    t = jnp.swapaxes(data, 1, 2)
    t2 = t.reshape(_C, _PIX)
    s = jnp.sum(t2, axis=0, keepdims=True)
    o_ref[...] = t2 / s


def kernel(x):
    y = pl.pallas_call(
        _body,
        grid=(_B,),
        in_specs=[pl.BlockSpec((1, _C, _W, _H), lambda b: (b, 0, 0, 0))],
        out_specs=pl.BlockSpec((_C, _PIX), lambda b: (0, b)),
        out_shape=jax.ShapeDtypeStruct((_C, _B * _PIX), jnp.float32),
        compiler_params=pltpu.CompilerParams(
            dimension_semantics=("arbitrary",),
        ),
    )(x)
    return y.T


# speedup vs baseline: 17.9256x; 1.2780x over previous
import jax
import jax.numpy as jnp
from jax.experimental import pallas as pl
from jax.experimental.pallas import tpu as pltpu

_B, _C, _W, _H = 32, 19, 128, 128
_PIX = _W * _H


_IB = 2


def _body(x_ref, o_ref):
    for i in range(_IB):
        data = x_ref[i]
        t = jnp.swapaxes(data, 1, 2)
        t2 = t.reshape(_C, _PIX)
        s = jnp.sum(t2, axis=0, keepdims=True)
        o_ref[:, i * _PIX:(i + 1) * _PIX] = t2 / s


def kernel(x):
    y = pl.pallas_call(
        _body,
        grid=(_B // _IB,),
        in_specs=[pl.BlockSpec((_IB, _C, _W, _H), lambda b: (b, 0, 0, 0))],
        out_specs=pl.BlockSpec((_C, _IB * _PIX), lambda b: (0, b)),
        out_shape=jax.ShapeDtypeStruct((_C, _B * _PIX), jnp.float32),
        compiler_params=pltpu.CompilerParams(
            dimension_semantics=("arbitrary",),
        ),
    )(x)
    return y.T


# speedup vs baseline: 19.3917x; 1.0818x over previous
import jax
import jax.numpy as jnp
from jax.experimental import pallas as pl
from jax.experimental.pallas import tpu as pltpu

_B, _C, _W, _H = 32, 19, 128, 128
_PIX = _W * _H


_IB = 4


def _body(x_ref, o_ref):
    for i in range(_IB):
        data = x_ref[i]
        t = jnp.swapaxes(data, 1, 2)
        t2 = t.reshape(_C, _PIX)
        s = jnp.sum(t2, axis=0, keepdims=True)
        o_ref[:, i * _PIX:(i + 1) * _PIX] = t2 / s


def kernel(x):
    y = pl.pallas_call(
        _body,
        grid=(_B // _IB,),
        in_specs=[pl.BlockSpec((_IB, _C, _W, _H), lambda b: (b, 0, 0, 0))],
        out_specs=pl.BlockSpec((_C, _IB * _PIX), lambda b: (0, b)),
        out_shape=jax.ShapeDtypeStruct((_C, _B * _PIX), jnp.float32),
        compiler_params=pltpu.CompilerParams(
            dimension_semantics=("arbitrary",),
        ),
    )(x)
    return y.T


# speedup vs baseline: 19.5168x; 1.0065x over previous
import jax
import jax.numpy as jnp
from jax.experimental import pallas as pl
from jax.experimental.pallas import tpu as pltpu

_B, _C, _W, _H = 32, 19, 128, 128
_PIX = _W * _H


_IB = 8


def _body(x_ref, o_ref):
    for i in range(_IB):
        data = x_ref[i]
        t = jnp.swapaxes(data, 1, 2)
        t2 = t.reshape(_C, _PIX)
        s = jnp.sum(t2, axis=0, keepdims=True)
        o_ref[:, i * _PIX:(i + 1) * _PIX] = t2 / s


def kernel(x):
    y = pl.pallas_call(
        _body,
        grid=(_B // _IB,),
        in_specs=[pl.BlockSpec((_IB, _C, _W, _H), lambda b: (b, 0, 0, 0))],
        out_specs=pl.BlockSpec((_C, _IB * _PIX), lambda b: (0, b)),
        out_shape=jax.ShapeDtypeStruct((_C, _B * _PIX), jnp.float32),
        compiler_params=pltpu.CompilerParams(
            dimension_semantics=("arbitrary",),
        ),
    )(x)
    return y.T
